# baseline (speedup 1.0000x reference)
import dataclasses
import functools

import jax
import jax.numpy as jnp
from jax import lax
from jax.experimental import pallas as pl
from jax.experimental.pallas import tpu as pltpu
from jax.experimental.pallas import tpu_sc as plsc

N = 10000
E = 320000
K = 128
NUM_WORKERS = 32
CPW = 80
E_PAD = NUM_WORKERS * CPW * K
NCHUNKS = E_PAD // K
ROWS_PER_SUBCORE = 624
ACC_ROWS = N + 8


HALF = CPW // 2


def _sc_agg_body(sup_hbm, idx_hbm, zeros_hbm, out_hbm,
                 idxall, rows0, rows1, acc, sg0, sg1):
    cid = lax.axis_index("c")
    sid = lax.axis_index("s")
    w = cid * 16 + sid

    r0 = sid * ROWS_PER_SUBCORE
    pltpu.sync_copy(zeros_hbm.at[pl.ds(r0, ROWS_PER_SUBCORE)],
                    acc.at[pl.ds(r0, ROWS_PER_SUBCORE)])

    tail0 = 16 * ROWS_PER_SUBCORE

    @pl.when(sid == 0)
    def _():
        pltpu.sync_copy(zeros_hbm.at[pl.ds(tail0, N - tail0)],
                        acc.at[pl.ds(tail0, N - tail0)])

    plsc.subcore_barrier()

    def gather(i, rows, sem):
        pltpu.async_copy(sup_hbm.at[idxall.at[2 * i]], rows, sem)

    def wait(rows, sem):
        pltpu.make_async_copy(sup_hbm.at[pl.ds(0, K)], rows, sem).wait()

    def scatter(i, rows):
        pltpu.sync_copy(rows, acc.at[idxall.at[2 * i + 1]], add=True)

    for h in range(2):
        pltpu.sync_copy(idx_hbm.at[pl.ds((w * 2 + h) * CPW, CPW)], idxall)
        gather(0, rows0, sg0)

        @pl.loop(0, HALF, step=2)
        def _(c):
            gather(c + 1, rows1, sg1)
            wait(rows0, sg0)
            scatter(c, rows0)

            @pl.when(c + 2 < HALF)
            def _():
                gather(c + 2, rows0, sg0)

            wait(rows1, sg1)
            scatter(c + 1, rows1)

    plsc.subcore_barrier()
    pltpu.sync_copy(acc.at[pl.ds(r0, ROWS_PER_SUBCORE)],
                    out_hbm.at[cid, pl.ds(r0, ROWS_PER_SUBCORE)])

    @pl.when(sid == 0)
    def _():
        pltpu.sync_copy(acc.at[pl.ds(tail0, N - tail0)],
                        out_hbm.at[cid, pl.ds(tail0, N - tail0)])


@functools.lru_cache(maxsize=None)
def _make_sc_agg(d):
    mesh = plsc.VectorSubcoreMesh(core_axis_name="c", subcore_axis_name="s")
    cp = pltpu.CompilerParams(use_tc_tiling_on_sc=False)
    return pl.kernel(
        _sc_agg_body,
        out_type=jax.ShapeDtypeStruct((2, N, d), jnp.float32),
        mesh=mesh,
        compiler_params=cp,
        scratch_types=[
            pltpu.VMEM((CPW, K), jnp.int32),
            pltpu.VMEM((K, d), jnp.float32),
            pltpu.VMEM((K, d), jnp.float32),
            pltpu.VMEM_SHARED((ACC_ROWS, d), jnp.float32),
            pltpu.SemaphoreType.DMA,
            pltpu.SemaphoreType.DMA,
        ],
    )


NP = 10240


def _sc_agg1_body(sup_hbm, idx_hbm, out_hbm, idxall, sup_v, acc_v):
    cid = lax.axis_index("c")
    sid = lax.axis_index("s")
    w = cid * 16 + sid

    pltpu.sync_copy(idx_hbm.at[pl.ds(w * 2 * CPW, 2 * CPW)], idxall)
    pltpu.sync_copy(sup_hbm, sup_v)

    @pl.loop(0, NP, step=16)
    def _(i):
        acc_v[pl.ds(i, 16)] = jnp.zeros((16,), jnp.float32)

    @pl.loop(0, CPW)
    def _(i):
        @pl.loop(0, K, step=16)
        def _(j):
            sv = idxall[2 * i, pl.ds(j, 16)]
            dv = idxall[2 * i + 1, pl.ds(j, 16)]
            vals = plsc.load_gather(sup_v, [sv])
            plsc.addupdate_scatter(acc_v, [dv], vals)

    pltpu.sync_copy(acc_v, out_hbm.at[pl.ds(w * NP, NP)])


def _make_sc_agg1():
    mesh = plsc.VectorSubcoreMesh(core_axis_name="c", subcore_axis_name="s")
    cp = pltpu.CompilerParams()
    if "needs_layout_passes" in pltpu.CompilerParams.__dataclass_fields__:
        cp = dataclasses.replace(cp, needs_layout_passes=False)
    return pl.kernel(
        _sc_agg1_body,
        out_type=jax.ShapeDtypeStruct((NUM_WORKERS * NP,), jnp.float32),
        mesh=mesh,
        compiler_params=cp,
        scratch_types=[
            pltpu.VMEM((2 * CPW, K), jnp.int32),
            pltpu.VMEM((NP,), jnp.float32),
            pltpu.VMEM((NP,), jnp.float32),
        ],
    )


def _mm_plain_body(x_ref, w_ref, o_ref):
    o_ref[...] = jnp.dot(x_ref[...], w_ref[...],
                         preferred_element_type=jnp.float32)


def _mm_fused_body(p_ref, b_ref, w_ref, o_ref):
    x = jnp.maximum(p_ref[0] + p_ref[1] + b_ref[...], 0.0)
    o_ref[...] = jnp.dot(x, w_ref[...], preferred_element_type=jnp.float32)


def _final_body(p_ref, b_ref, o_ref):
    row = jnp.sum(p_ref[...], axis=0, keepdims=True) + b_ref[0, 0]
    mx = jnp.max(row)
    e = jnp.exp(row - mx)
    s = jnp.sum(e)
    o_ref[...] = row - (mx + jnp.log(s))


_BM = 1000


def _mm_plain(x, w):
    dout = w.shape[1]
    return pl.pallas_call(
        _mm_plain_body,
        grid=(N // _BM,),
        in_specs=[pl.BlockSpec((_BM, x.shape[1]), lambda i: (i, 0)),
                  pl.BlockSpec((x.shape[1], dout), lambda i: (0, 0))],
        out_specs=pl.BlockSpec((_BM, dout), lambda i: (i, 0)),
        out_shape=jax.ShapeDtypeStruct((N, dout), jnp.float32),
    )(x, w)


def _mm_fused(p, b, w):
    din = p.shape[2]
    dout = w.shape[1]
    return pl.pallas_call(
        _mm_fused_body,
        grid=(N // _BM,),
        in_specs=[pl.BlockSpec((2, _BM, din), lambda i: (0, i, 0)),
                  pl.BlockSpec((1, din), lambda i: (0, 0)),
                  pl.BlockSpec((din, dout), lambda i: (0, 0))],
        out_specs=pl.BlockSpec((_BM, dout), lambda i: (i, 0)),
        out_shape=jax.ShapeDtypeStruct((N, dout), jnp.float32),
    )(p, b, w)


def _final(p32, b):
    return pl.pallas_call(
        _final_body,
        in_specs=[pl.BlockSpec((NUM_WORKERS, N), lambda: (0, 0)),
                  pl.BlockSpec((1, 1), lambda: (0, 0))],
        out_specs=pl.BlockSpec((1, N), lambda: (0, 0)),
        out_shape=jax.ShapeDtypeStruct((1, N), jnp.float32),
    )(p32, b)


def kernel(features, adj_matrix, W1, b1, W2, b2, W3, b3, W4, b4, W5, b5):
    pad = E_PAD - E
    src = jnp.concatenate([adj_matrix[0], jnp.zeros((pad,), jnp.int32)])
    dst = jnp.concatenate([adj_matrix[1], jnp.full((pad,), N, jnp.int32)])
    idx = jnp.stack([src.reshape(NCHUNKS, K), dst.reshape(NCHUNKS, K)],
                    axis=1).reshape(2 * NCHUNKS, K)

    zeros128 = jnp.zeros((N, 128), jnp.float32)
    b1r = b1.reshape(1, 128)
    b2r = b2.reshape(1, 128)
    b3r = b3.reshape(1, 128)
    b4r = b4.reshape(1, 128)
    b5r = b5.reshape(1, 1)

    agg128 = _make_sc_agg(128)
    agg1 = _make_sc_agg1()

    s = _mm_plain(features, W1)
    p = agg128(s, idx, zeros128)
    s = _mm_fused(p, b1r, W2)
    p = agg128(s, idx, zeros128)
    s = _mm_fused(p, b2r, W3)
    p = agg128(s, idx, zeros128)
    s = _mm_fused(p, b3r, W4)
    p = agg128(s, idx, zeros128)
    s = _mm_fused(p, b4r, W5)
    s5p = jnp.pad(s.reshape(-1), (0, NP - N))
    p32 = agg1(s5p, idx).reshape(NUM_WORKERS, NP)[:, :N]
    out = _final(p32, b5r)
    return out.reshape(-1)


# speedup vs baseline: 1.0014x; 1.0014x over previous
import dataclasses
import functools

import jax
import jax.numpy as jnp
from jax import lax
from jax.experimental import pallas as pl
from jax.experimental.pallas import tpu as pltpu
from jax.experimental.pallas import tpu_sc as plsc

N = 10000
E = 320000
K = 128
NUM_WORKERS = 32
CPW = 80
E_PAD = NUM_WORKERS * CPW * K
NCHUNKS = E_PAD // K
ROWS_PER_SUBCORE = 624
ACC_ROWS = N + 8


HALF = CPW // 2


def _sc_agg_body(sup_hbm, idx_hbm, zeros_hbm, out_hbm,
                 idxall, rows0, rows1, acc, sg0, sg1):
    cid = lax.axis_index("c")
    sid = lax.axis_index("s")
    w = cid * 16 + sid

    r0 = sid * ROWS_PER_SUBCORE
    pltpu.sync_copy(zeros_hbm.at[pl.ds(r0, ROWS_PER_SUBCORE)],
                    acc.at[pl.ds(r0, ROWS_PER_SUBCORE)])

    tail0 = 16 * ROWS_PER_SUBCORE

    @pl.when(sid == 0)
    def _():
        pltpu.sync_copy(zeros_hbm.at[pl.ds(tail0, N - tail0)],
                        acc.at[pl.ds(tail0, N - tail0)])

    plsc.subcore_barrier()

    def gather(i, rows, sem):
        pltpu.async_copy(sup_hbm.at[idxall.at[2 * i, pl.ds(0, K // 2)]],
                         rows.at[pl.ds(0, K // 2)], sem)
        pltpu.async_copy(sup_hbm.at[idxall.at[2 * i, pl.ds(K // 2, K // 2)]],
                         rows.at[pl.ds(K // 2, K // 2)], sem)

    def wait(rows, sem):
        pltpu.make_async_copy(sup_hbm.at[pl.ds(0, K)], rows, sem).wait()

    def scatter(i, rows):
        pltpu.sync_copy(rows, acc.at[idxall.at[2 * i + 1]], add=True)

    for h in range(2):
        pltpu.sync_copy(idx_hbm.at[pl.ds((w * 2 + h) * CPW, CPW)], idxall)
        gather(0, rows0, sg0)

        @pl.loop(0, HALF, step=2)
        def _(c):
            gather(c + 1, rows1, sg1)
            wait(rows0, sg0)
            scatter(c, rows0)

            @pl.when(c + 2 < HALF)
            def _():
                gather(c + 2, rows0, sg0)

            wait(rows1, sg1)
            scatter(c + 1, rows1)

    plsc.subcore_barrier()
    pltpu.sync_copy(acc.at[pl.ds(r0, ROWS_PER_SUBCORE)],
                    out_hbm.at[cid, pl.ds(r0, ROWS_PER_SUBCORE)])

    @pl.when(sid == 0)
    def _():
        pltpu.sync_copy(acc.at[pl.ds(tail0, N - tail0)],
                        out_hbm.at[cid, pl.ds(tail0, N - tail0)])


@functools.lru_cache(maxsize=None)
def _make_sc_agg(d):
    mesh = plsc.VectorSubcoreMesh(core_axis_name="c", subcore_axis_name="s")
    cp = pltpu.CompilerParams(use_tc_tiling_on_sc=False)
    return pl.kernel(
        _sc_agg_body,
        out_type=jax.ShapeDtypeStruct((2, N, d), jnp.float32),
        mesh=mesh,
        compiler_params=cp,
        scratch_types=[
            pltpu.VMEM((CPW, K), jnp.int32),
            pltpu.VMEM((K, d), jnp.float32),
            pltpu.VMEM((K, d), jnp.float32),
            pltpu.VMEM_SHARED((ACC_ROWS, d), jnp.float32),
            pltpu.SemaphoreType.DMA,
            pltpu.SemaphoreType.DMA,
        ],
    )


NP = 10240


def _sc_agg1_body(sup_hbm, idx_hbm, out_hbm, idxall, sup_v, acc_v):
    cid = lax.axis_index("c")
    sid = lax.axis_index("s")
    w = cid * 16 + sid

    pltpu.sync_copy(idx_hbm.at[pl.ds(w * 2 * CPW, 2 * CPW)], idxall)
    pltpu.sync_copy(sup_hbm, sup_v)

    @pl.loop(0, NP, step=16)
    def _(i):
        acc_v[pl.ds(i, 16)] = jnp.zeros((16,), jnp.float32)

    @pl.loop(0, CPW)
    def _(i):
        @pl.loop(0, K, step=16)
        def _(j):
            sv = idxall[2 * i, pl.ds(j, 16)]
            dv = idxall[2 * i + 1, pl.ds(j, 16)]
            vals = plsc.load_gather(sup_v, [sv])
            plsc.addupdate_scatter(acc_v, [dv], vals)

    pltpu.sync_copy(acc_v, out_hbm.at[pl.ds(w * NP, NP)])


def _make_sc_agg1():
    mesh = plsc.VectorSubcoreMesh(core_axis_name="c", subcore_axis_name="s")
    cp = pltpu.CompilerParams()
    if "needs_layout_passes" in pltpu.CompilerParams.__dataclass_fields__:
        cp = dataclasses.replace(cp, needs_layout_passes=False)
    return pl.kernel(
        _sc_agg1_body,
        out_type=jax.ShapeDtypeStruct((NUM_WORKERS * NP,), jnp.float32),
        mesh=mesh,
        compiler_params=cp,
        scratch_types=[
            pltpu.VMEM((2 * CPW, K), jnp.int32),
            pltpu.VMEM((NP,), jnp.float32),
            pltpu.VMEM((NP,), jnp.float32),
        ],
    )


def _mm_plain_body(x_ref, w_ref, o_ref):
    o_ref[...] = jnp.dot(x_ref[...], w_ref[...],
                         preferred_element_type=jnp.float32)


def _mm_fused_body(p_ref, b_ref, w_ref, o_ref):
    x = jnp.maximum(p_ref[0] + p_ref[1] + b_ref[...], 0.0)
    o_ref[...] = jnp.dot(x, w_ref[...], preferred_element_type=jnp.float32)


def _final_body(p_ref, b_ref, o_ref):
    row = jnp.sum(p_ref[...], axis=0, keepdims=True) + b_ref[0, 0]
    mx = jnp.max(row)
    e = jnp.exp(row - mx)
    s = jnp.sum(e)
    o_ref[...] = row - (mx + jnp.log(s))


_BM = 1000


def _mm_plain(x, w):
    dout = w.shape[1]
    return pl.pallas_call(
        _mm_plain_body,
        grid=(N // _BM,),
        in_specs=[pl.BlockSpec((_BM, x.shape[1]), lambda i: (i, 0)),
                  pl.BlockSpec((x.shape[1], dout), lambda i: (0, 0))],
        out_specs=pl.BlockSpec((_BM, dout), lambda i: (i, 0)),
        out_shape=jax.ShapeDtypeStruct((N, dout), jnp.float32),
    )(x, w)


def _mm_fused(p, b, w):
    din = p.shape[2]
    dout = w.shape[1]
    return pl.pallas_call(
        _mm_fused_body,
        grid=(N // _BM,),
        in_specs=[pl.BlockSpec((2, _BM, din), lambda i: (0, i, 0)),
                  pl.BlockSpec((1, din), lambda i: (0, 0)),
                  pl.BlockSpec((din, dout), lambda i: (0, 0))],
        out_specs=pl.BlockSpec((_BM, dout), lambda i: (i, 0)),
        out_shape=jax.ShapeDtypeStruct((N, dout), jnp.float32),
    )(p, b, w)


def _final(p32, b):
    return pl.pallas_call(
        _final_body,
        in_specs=[pl.BlockSpec((NUM_WORKERS, N), lambda: (0, 0)),
                  pl.BlockSpec((1, 1), lambda: (0, 0))],
        out_specs=pl.BlockSpec((1, N), lambda: (0, 0)),
        out_shape=jax.ShapeDtypeStruct((1, N), jnp.float32),
    )(p32, b)


def kernel(features, adj_matrix, W1, b1, W2, b2, W3, b3, W4, b4, W5, b5):
    pad = E_PAD - E
    src = jnp.concatenate([adj_matrix[0], jnp.zeros((pad,), jnp.int32)])
    dst = jnp.concatenate([adj_matrix[1], jnp.full((pad,), N, jnp.int32)])
    idx = jnp.stack([src.reshape(NCHUNKS, K), dst.reshape(NCHUNKS, K)],
                    axis=1).reshape(2 * NCHUNKS, K)

    zeros128 = jnp.zeros((N, 128), jnp.float32)
    b1r = b1.reshape(1, 128)
    b2r = b2.reshape(1, 128)
    b3r = b3.reshape(1, 128)
    b4r = b4.reshape(1, 128)
    b5r = b5.reshape(1, 1)

    agg128 = _make_sc_agg(128)
    agg1 = _make_sc_agg1()

    s = _mm_plain(features, W1)
    p = agg128(s, idx, zeros128)
    s = _mm_fused(p, b1r, W2)
    p = agg128(s, idx, zeros128)
    s = _mm_fused(p, b2r, W3)
    p = agg128(s, idx, zeros128)
    s = _mm_fused(p, b3r, W4)
    p = agg128(s, idx, zeros128)
    s = _mm_fused(p, b4r, W5)
    s5p = jnp.pad(s.reshape(-1), (0, NP - N))
    p32 = agg1(s5p, idx).reshape(NUM_WORKERS, NP)[:, :N]
    out = _final(p32, b5r)
    return out.reshape(-1)


# speedup vs baseline: 1.0089x; 1.0075x over previous
import dataclasses
import functools

import jax
import jax.numpy as jnp
from jax import lax
from jax.experimental import pallas as pl
from jax.experimental.pallas import tpu as pltpu
from jax.experimental.pallas import tpu_sc as plsc

N = 10000
E = 320000
K = 128
NUM_WORKERS = 32
CPW = 80
E_PAD = NUM_WORKERS * CPW * K
NCHUNKS = E_PAD // K
ROWS_PER_SUBCORE = 624
ACC_ROWS = N + 8


HALF = CPW // 2


def _sc_agg_body(sup_hbm, idx_hbm, zeros_hbm, out_hbm,
                 idxall, rows0, rows1, acc, sg0, sg1):
    cid = lax.axis_index("c")
    sid = lax.axis_index("s")
    w = cid * 16 + sid

    r0 = sid * ROWS_PER_SUBCORE
    pltpu.sync_copy(zeros_hbm.at[pl.ds(r0, ROWS_PER_SUBCORE)],
                    acc.at[pl.ds(r0, ROWS_PER_SUBCORE)])

    tail0 = 16 * ROWS_PER_SUBCORE

    @pl.when(sid == 0)
    def _():
        pltpu.sync_copy(zeros_hbm.at[pl.ds(tail0, N - tail0)],
                        acc.at[pl.ds(tail0, N - tail0)])

    plsc.subcore_barrier()

    def gather(i, rows, sem):
        pltpu.async_copy(sup_hbm.at[idxall.at[2 * i]], rows, sem)

    def wait(rows, sem):
        pltpu.make_async_copy(sup_hbm.at[pl.ds(0, K)], rows, sem).wait()

    def scatter(i, rows):
        pltpu.sync_copy(rows, acc.at[idxall.at[2 * i + 1]], add=True)

    for h in range(2):
        pltpu.sync_copy(idx_hbm.at[pl.ds((w * 2 + h) * CPW, CPW)], idxall)
        gather(0, rows0, sg0)

        @pl.loop(0, HALF, step=2)
        def _(c):
            gather(c + 1, rows1, sg1)
            wait(rows0, sg0)
            scatter(c, rows0)

            @pl.when(c + 2 < HALF)
            def _():
                gather(c + 2, rows0, sg0)

            wait(rows1, sg1)
            scatter(c + 1, rows1)

    plsc.subcore_barrier()
    pltpu.sync_copy(acc.at[pl.ds(r0, ROWS_PER_SUBCORE)],
                    out_hbm.at[cid, pl.ds(r0, ROWS_PER_SUBCORE)])

    @pl.when(sid == 0)
    def _():
        pltpu.sync_copy(acc.at[pl.ds(tail0, N - tail0)],
                        out_hbm.at[cid, pl.ds(tail0, N - tail0)])


@functools.lru_cache(maxsize=None)
def _make_sc_agg(d):
    mesh = plsc.VectorSubcoreMesh(core_axis_name="c", subcore_axis_name="s")
    cp = pltpu.CompilerParams(use_tc_tiling_on_sc=False)
    return pl.kernel(
        _sc_agg_body,
        out_type=jax.ShapeDtypeStruct((2, N, d), jnp.float32),
        mesh=mesh,
        compiler_params=cp,
        scratch_types=[
            pltpu.VMEM((CPW, K), jnp.int32),
            pltpu.VMEM((K, d), jnp.float32),
            pltpu.VMEM((K, d), jnp.float32),
            pltpu.VMEM_SHARED((ACC_ROWS, d), jnp.float32),
            pltpu.SemaphoreType.DMA,
            pltpu.SemaphoreType.DMA,
        ],
    )


NP = 10240


def _sc_agg1_body(sup_hbm, idx_hbm, out_hbm, idxall, sup_v, acc_v):
    cid = lax.axis_index("c")
    sid = lax.axis_index("s")
    w = cid * 16 + sid

    pltpu.sync_copy(idx_hbm.at[pl.ds(w * 2 * CPW, 2 * CPW)], idxall)
    pltpu.sync_copy(sup_hbm, sup_v)

    @pl.loop(0, NP, step=16)
    def _(i):
        acc_v[pl.ds(i, 16)] = jnp.zeros((16,), jnp.float32)

    @pl.loop(0, CPW)
    def _(i):
        @pl.loop(0, K, step=16)
        def _(j):
            sv = idxall[2 * i, pl.ds(j, 16)]
            dv = idxall[2 * i + 1, pl.ds(j, 16)]
            vals = plsc.load_gather(sup_v, [sv])
            plsc.addupdate_scatter(acc_v, [dv], vals)

    pltpu.sync_copy(acc_v, out_hbm.at[pl.ds(w * NP, NP)])


def _make_sc_agg1():
    mesh = plsc.VectorSubcoreMesh(core_axis_name="c", subcore_axis_name="s")
    cp = pltpu.CompilerParams()
    if "needs_layout_passes" in pltpu.CompilerParams.__dataclass_fields__:
        cp = dataclasses.replace(cp, needs_layout_passes=False)
    return pl.kernel(
        _sc_agg1_body,
        out_type=jax.ShapeDtypeStruct((NUM_WORKERS * NP,), jnp.float32),
        mesh=mesh,
        compiler_params=cp,
        scratch_types=[
            pltpu.VMEM((2 * CPW, K), jnp.int32),
            pltpu.VMEM((NP,), jnp.float32),
            pltpu.VMEM((NP,), jnp.float32),
        ],
    )


def _mm_plain_body(x_ref, w_ref, o_ref):
    o_ref[...] = jnp.dot(x_ref[...], w_ref[...],
                         preferred_element_type=jnp.float32)


def _mm_fused_body(p_ref, b_ref, w_ref, o_ref):
    x = jnp.maximum(p_ref[0] + p_ref[1] + b_ref[...], 0.0)
    o_ref[...] = jnp.dot(x, w_ref[...], preferred_element_type=jnp.float32)


def _final_body(p_ref, b_ref, o_ref):
    row = jnp.sum(p_ref[...], axis=0, keepdims=True) + b_ref[0, 0]
    mx = jnp.max(row)
    e = jnp.exp(row - mx)
    s = jnp.sum(e)
    o_ref[...] = row - (mx + jnp.log(s))


_BM = 2000


def _mm_plain(x, w):
    dout = w.shape[1]
    return pl.pallas_call(
        _mm_plain_body,
        grid=(N // _BM,),
        in_specs=[pl.BlockSpec((_BM, x.shape[1]), lambda i: (i, 0)),
                  pl.BlockSpec((x.shape[1], dout), lambda i: (0, 0))],
        out_specs=pl.BlockSpec((_BM, dout), lambda i: (i, 0)),
        out_shape=jax.ShapeDtypeStruct((N, dout), jnp.float32),
    )(x, w)


def _mm_fused(p, b, w):
    din = p.shape[2]
    dout = w.shape[1]
    return pl.pallas_call(
        _mm_fused_body,
        grid=(N // _BM,),
        in_specs=[pl.BlockSpec((2, _BM, din), lambda i: (0, i, 0)),
                  pl.BlockSpec((1, din), lambda i: (0, 0)),
                  pl.BlockSpec((din, dout), lambda i: (0, 0))],
        out_specs=pl.BlockSpec((_BM, dout), lambda i: (i, 0)),
        out_shape=jax.ShapeDtypeStruct((N, dout), jnp.float32),
    )(p, b, w)


def _final(p32, b):
    return pl.pallas_call(
        _final_body,
        in_specs=[pl.BlockSpec((NUM_WORKERS, N), lambda: (0, 0)),
                  pl.BlockSpec((1, 1), lambda: (0, 0))],
        out_specs=pl.BlockSpec((1, N), lambda: (0, 0)),
        out_shape=jax.ShapeDtypeStruct((1, N), jnp.float32),
    )(p32, b)


def kernel(features, adj_matrix, W1, b1, W2, b2, W3, b3, W4, b4, W5, b5):
    pad = E_PAD - E
    src = jnp.concatenate([adj_matrix[0], jnp.zeros((pad,), jnp.int32)])
    dst = jnp.concatenate([adj_matrix[1], jnp.full((pad,), N, jnp.int32)])
    idx = jnp.stack([src.reshape(NCHUNKS, K), dst.reshape(NCHUNKS, K)],
                    axis=1).reshape(2 * NCHUNKS, K)

    zeros128 = jnp.zeros((N, 128), jnp.float32)
    b1r = b1.reshape(1, 128)
    b2r = b2.reshape(1, 128)
    b3r = b3.reshape(1, 128)
    b4r = b4.reshape(1, 128)
    b5r = b5.reshape(1, 1)

    agg128 = _make_sc_agg(128)
    agg1 = _make_sc_agg1()

    s = _mm_plain(features, W1)
    p = agg128(s, idx, zeros128)
    s = _mm_fused(p, b1r, W2)
    p = agg128(s, idx, zeros128)
    s = _mm_fused(p, b2r, W3)
    p = agg128(s, idx, zeros128)
    s = _mm_fused(p, b3r, W4)
    p = agg128(s, idx, zeros128)
    s = _mm_fused(p, b4r, W5)
    s5p = jnp.pad(s.reshape(-1), (0, NP - N))
    p32 = agg1(s5p, idx).reshape(NUM_WORKERS, NP)[:, :N]
    out = _final(p32, b5r)
    return out.reshape(-1)
